# baseline (speedup 1.0000x reference)
import functools

import jax
import jax.numpy as jnp
from jax import lax
from jax.experimental import pallas as pl
from jax.experimental.pallas import tpu as pltpu
from jax.experimental.pallas import tpu_sc as plsc

S = 512
D = 768
R = 2 * S - 1
NW = 32
RPW = S // NW
G = 16
BLOCKS = [(48 * t, 48) for t in range(10)] + [(480, 32)]
SPAN = 8 * (G - 1)
CHUNK = SPAN + 48
RPAD = 1032


def kernel(embedding, seq_len):
    del seq_len

    emb8 = jnp.concatenate(
        [
            jnp.pad(embedding, ((k, RPAD - R - k), (0, 0)))
            for k in range(8)
        ],
        axis=0,
    ).reshape(8, RPAD, D)

    mesh = plsc.VectorSubcoreMesh(core_axis_name="c", subcore_axis_name="s")

    @functools.partial(
        pl.kernel,
        mesh=mesh,
        out_type=jax.ShapeDtypeStruct((S, S, D), jnp.float32),
        scratch_types=[
            pltpu.VMEM((CHUNK, D), jnp.float32),
            pltpu.SemaphoreType.DMA,
            pltpu.SemaphoreType.DMA,
            pltpu.SemaphoreType.DMA,
            pltpu.SemaphoreType.DMA,
        ],
    )
    def sliding_copy(
        emb_hbm, out_hbm, chunk_v, load_sem, sem_conf, sem_f0, sem_f1
    ):
        wid = lax.axis_index("s") * 2 + lax.axis_index("c")
        res = lax.rem(wid, 8)
        q = wid // 8
        k = lax.rem(res + 1, 8)

        i_base = res + 8 * (RPW * q)
        T_base = pl.multiple_of(-i_base + (S - 1 - SPAN) + k, 8)

        def ring_load(x0, ln):
            handles = []
            pos = x0 % CHUNK
            for (rp, off, n) in (
                [(pos, 0, min(ln, CHUNK - pos))]
                + ([(0, CHUNK - pos, pos + ln - CHUNK)]
                   if pos + ln > CHUNK else [])
            ):
                handles.append(
                    pltpu.async_copy(
                        emb_hbm.at[k, pl.ds(T_base + x0 + off, n), :],
                        chunk_v.at[pl.ds(rp, n)],
                        load_sem,
                    )
                )
            return handles

        sems_free = (sem_f0, sem_f1)

        def fire_store(j0, w, m, sem):
            handles = []
            x0 = j0 + 8 * (G - 1 - m)
            pos = x0 % CHUNK
            for (rp, off, n) in (
                [(pos, 0, min(w, CHUNK - pos))]
                + ([(0, CHUNK - pos, pos + w - CHUNK)]
                   if pos + w > CHUNK else [])
            ):
                handles.append(
                    pltpu.async_copy(
                        chunk_v.at[pl.ds(rp, n), :],
                        out_hbm.at[i_base + 8 * m, pl.ds(j0 + off, n), :],
                        sem,
                    )
                )
            return handles

        pend_conf = []
        pend_free = {0: [], 1: []}
        for t, (j0, w) in enumerate(BLOCKS):
            w_next = BLOCKS[t + 1][1] if t + 1 < len(BLOCKS) else None
            conf_cut = (G - w_next // 8) if w_next is not None else G + 1
            new_conf, new_free = [], []

            def fire(m):
                hs = fire_store(
                    j0, w, m,
                    sem_conf if m >= conf_cut else sems_free[t & 1],
                )
                (new_conf if m >= conf_cut else new_free).extend(hs)

            if t == 0:
                for hd in ring_load(0, CHUNK):
                    hd.wait()
                for m in range(G):
                    fire(m)
            else:
                for hd in pend_conf:
                    hd.wait()
                for hd in pend_free[t & 1]:
                    hd.wait()
                pend_free[t & 1] = []
                load_h = ring_load(j0 + SPAN, w)
                cut = G - 1 - (SPAN - w) // 8
                for m in range(cut, G):
                    fire(m)
                for hd in load_h:
                    hd.wait()
                for m in range(cut):
                    fire(m)
            pend_conf = new_conf
            pend_free[t & 1].extend(new_free)
        for hd in pend_conf:
            hd.wait()
        for b in (0, 1):
            for hd in pend_free[b]:
                hd.wait()

    return sliding_copy(emb8)


# speedup vs baseline: 1.0127x; 1.0127x over previous
import functools

import jax
import jax.numpy as jnp
from jax import lax
from jax.experimental import pallas as pl
from jax.experimental.pallas import tpu as pltpu
from jax.experimental.pallas import tpu_sc as plsc

S = 512
D = 768
R = 2 * S - 1
NW = 32
RPW = S // NW
G = 16
BLOCKS = [(48 * t, 48) for t in range(10)] + [(480, 32)]
SPAN = 8 * (G - 1)
CHUNK = SPAN + 48
RPAD = 1032


def kernel(embedding, seq_len):
    del seq_len

    emb8 = jnp.concatenate(
        [
            jnp.pad(embedding, ((k, RPAD - R - k), (0, 0)))
            for k in range(8)
        ],
        axis=0,
    ).reshape(8, RPAD, D)

    mesh = plsc.VectorSubcoreMesh(core_axis_name="c", subcore_axis_name="s")

    @functools.partial(
        pl.kernel,
        mesh=mesh,
        out_type=jax.ShapeDtypeStruct((S, S, D), jnp.float32),
        scratch_types=[
            pltpu.VMEM((CHUNK, D), jnp.float32),
            pltpu.SemaphoreType.DMA,
            pltpu.SemaphoreType.DMA,
        ],
    )
    def sliding_copy(emb_hbm, out_hbm, chunk_v, load_sem, store_sem):
        wid = lax.axis_index("s") * 2 + lax.axis_index("c")
        res = lax.rem(wid, 8)
        q = wid // 8
        k = lax.rem(res + 1, 8)

        i_base = res + 8 * (RPW * q)
        T_base = pl.multiple_of(-i_base + (S - 1 - SPAN) + k, 8)

        def ring_load(x0, ln):
            handles = []
            pos = x0 % CHUNK
            for (rp, off, n) in (
                [(pos, 0, min(ln, CHUNK - pos))]
                + ([(0, CHUNK - pos, pos + ln - CHUNK)]
                   if pos + ln > CHUNK else [])
            ):
                handles.append(
                    pltpu.async_copy(
                        emb_hbm.at[k, pl.ds(T_base + x0 + off, n), :],
                        chunk_v.at[pl.ds(rp, n)],
                        load_sem,
                    )
                )
            return handles

        def fire_store(j0, w, m):
            handles = []
            x0 = j0 + 8 * (G - 1 - m)
            pos = x0 % CHUNK
            for (rp, off, n) in (
                [(pos, 0, min(w, CHUNK - pos))]
                + ([(0, CHUNK - pos, pos + w - CHUNK)]
                   if pos + w > CHUNK else [])
            ):
                handles.append(
                    pltpu.async_copy(
                        chunk_v.at[pl.ds(rp, n), :],
                        out_hbm.at[i_base + 8 * m, pl.ds(j0 + off, n), :],
                        store_sem,
                    )
                )
            return handles

        for hd in ring_load(0, CHUNK):
            hd.wait()
        first = True
        for (j0, w) in BLOCKS:
            handles = []
            if first:
                for m in range(G):
                    handles += fire_store(j0, w, m)
                first = False
            else:
                load_h = ring_load(j0 + SPAN, w)
                cut = G - 1 - (SPAN - w) // 8
                for m in range(cut, G):
                    handles += fire_store(j0, w, m)
                for hd in load_h:
                    hd.wait()
                for m in range(cut):
                    handles += fire_store(j0, w, m)
            for hd in handles:
                hd.wait()

    return sliding_copy(emb8)
